# baseline (speedup 1.0000x reference)
import jax
import jax.numpy as jnp
from jax.experimental import pallas as pl
from jax.experimental.pallas import tpu as pltpu

B, S, DIM = 1, 2048, 768
HEADS, HDIM = 12, 64
INTER = 1024
E, TOPK = 64, 2
EPS = 1e-6

SBLK = 512
NSB = S // SBLK
QBLK = 512
ABLK = 512
NAB = S // ABLK
T = 128
NA = S * TOPK
NB = NA // T + (E - 1)
CH = 256
NCH = NA // CH


def _rms(x, w):
    return w * (x * jax.lax.rsqrt(jnp.mean(x * x, axis=-1, keepdims=True) + EPS))


def _bdot(a, b):
    return jnp.dot(a.astype(jnp.bfloat16), b.astype(jnp.bfloat16),
                   preferred_element_type=jnp.float32)


def _f32i(x):
    return jnp.rint(x).astype(jnp.int32)


_HI = jax.lax.Precision.HIGHEST


def _qkv_kernel(x_ref, w1_ref, wq_ref, wk_ref, wv_ref, q_ref, k_ref, v_ref):
    xn = _rms(x_ref[...], w1_ref[...])
    q_ref[...] = _bdot(xn, wq_ref[...])
    k_ref[...] = _bdot(xn, wk_ref[...])
    v_ref[...] = _bdot(xn, wv_ref[...])


HG = 12


def _attn_kernel(q_ref, k_ref, v_ref, o_ref):
    for hp in range(HG):
        sl = slice(hp * HDIM, (hp + 1) * HDIM)
        q = (q_ref[:, sl] * (HDIM ** -0.5)).astype(jnp.bfloat16)
        k = k_ref[:, sl].astype(jnp.bfloat16)
        s = jax.lax.dot_general(q, k, (((1,), (1,)), ((), ())),
                                preferred_element_type=jnp.float32)
        m = jnp.max(s, axis=-1, keepdims=True)
        p = jnp.exp(s - m)
        denom = jnp.sum(p, axis=-1, keepdims=True)
        o_ref[:, sl] = _bdot(p, v_ref[:, sl]) / denom


def _route_kernel(x_ref, a_ref, wo_ref, w2_ref, wr_ref,
                  x1_ref, h_ref, st_ref, sw_ref, be_ref, br_ref, bn_ref,
                  i1s, i2s, was, wbs):
    i = pl.program_id(0)
    x1 = x_ref[...] + _bdot(a_ref[...], wo_ref[...])
    x1_ref[...] = x1
    h = _rms(x1, w2_ref[...])
    h_ref[...] = h

    logits = jnp.dot(h, wr_ref[...], preferred_element_type=jnp.float32)
    m = jnp.max(logits, axis=-1, keepdims=True)
    p = jnp.exp(logits - m)
    p = p / jnp.sum(p, axis=-1, keepdims=True)
    v1 = jnp.max(p, axis=-1, keepdims=True)
    i1 = jnp.argmax(p, axis=-1, keepdims=True).astype(jnp.int32)
    cols = jax.lax.broadcasted_iota(jnp.int32, (SBLK, E), 1)
    p2 = jnp.where(cols == i1, -1.0, p)
    v2 = jnp.max(p2, axis=-1, keepdims=True)
    i2 = jnp.argmax(p2, axis=-1, keepdims=True).astype(jnp.int32)
    tot = v1 + v2
    sl = pl.ds(i * SBLK, SBLK)
    i1s[sl, :] = i1
    i2s[sl, :] = i2
    was[sl, :] = v1 / tot
    wbs[sl, :] = v2 / tot

    @pl.when(i == NSB - 1)
    def _dispatch():
        _dispatch_body(i1s, i2s, was, wbs, st_ref, sw_ref,
                       be_ref, br_ref, bn_ref)


def _dispatch_body(i1_ref, i2_ref, wa_ref, wb_ref,
                   st_ref, sw_ref, be_ref, br_ref, bn_ref):
    eid = jnp.concatenate([i1_ref[...], i2_ref[...]], axis=0)
    wt = jnp.concatenate([wa_ref[...], wb_ref[...]], axis=0)
    tokf = jnp.remainder(
        jax.lax.broadcasted_iota(jnp.int32, (NA, 1), 0), S).astype(jnp.float32)

    onehot = (jax.lax.broadcasted_iota(jnp.int32, (NA, E), 1)
              == eid).astype(jnp.float32)
    counts_row = jnp.sum(onehot, axis=0, keepdims=True)

    r_i = jax.lax.broadcasted_iota(jnp.int32, (CH, CH), 0)
    c_i = jax.lax.broadcasted_iota(jnp.int32, (CH, CH), 1)
    stril = (c_i < r_i).astype(jnp.float32)
    carry = jnp.zeros((1, E), jnp.float32)
    rank_chunks = []
    for c in range(NCH):
        oc = onehot[c * CH:(c + 1) * CH, :]
        loc = _bdot(stril, oc)
        rank_chunks.append(jnp.sum((loc + carry) * oc, axis=1, keepdims=True))
        carry = carry + jnp.sum(oc, axis=0, keepdims=True)
    rank = jnp.concatenate(rank_chunks, axis=0)

    ue_r = jax.lax.broadcasted_iota(jnp.int32, (E, E), 0)
    ue_c = jax.lax.broadcasted_iota(jnp.int32, (E, E), 1)
    offs_row = jnp.dot(counts_row, (ue_r < ue_c).astype(jnp.float32),
                       preferred_element_type=jnp.float32, precision=_HI)
    dest = _f32i(jnp.sum(onehot * offs_row, axis=1, keepdims=True) + rank)

    tok_hi = jnp.floor(tokf / 64.0)
    tok_lo = tokf - tok_hi * 64.0
    wt_hi = wt.astype(jnp.bfloat16).astype(jnp.float32)
    wt_lo = wt - wt_hi
    vals = jnp.concatenate([tok_hi, tok_lo, wt_hi, wt_lo], axis=1)
    for c in range(NCH):
        r_row = (jax.lax.broadcasted_iota(jnp.int32, (1, CH), 1) + c * CH)
        mc = (dest == r_row).astype(jnp.float32)
        chunk = jax.lax.dot_general(
            mc.astype(jnp.bfloat16), vals.astype(jnp.bfloat16),
            (((0,), (0,)), ((), ())),
            preferred_element_type=jnp.float32)
        st_ref[c * CH:(c + 1) * CH, :] = _f32i(chunk[:, 0:1] * 64.0
                                               + chunk[:, 1:2])
        sw_ref[c * CH:(c + 1) * CH, :] = chunk[:, 2:3] + chunk[:, 3:4]
    st_ref[NA:, :] = jnp.zeros((T, 1), jnp.int32)
    sw_ref[NA:, :] = jnp.zeros((T, 1), jnp.float32)

    counts_i = _f32i(counts_row)
    ntiles_i = (counts_i + (T - 1)) // T
    ctiles_row = jnp.dot(ntiles_i.astype(jnp.float32),
                         (ue_r <= ue_c).astype(jnp.float32),
                         preferred_element_type=jnp.float32,
                         precision=_HI)
    ctiles_i = _f32i(ctiles_row)
    b_col = jax.lax.broadcasted_iota(jnp.int32, (NB, 1), 0)
    bexp = jnp.minimum(jnp.sum((ctiles_i <= b_col).astype(jnp.int32),
                               axis=1, keepdims=True), E - 1)
    oh_b = (jax.lax.broadcasted_iota(jnp.int32, (NB, E), 1) == bexp)
    g_ct = jnp.sum(jnp.where(oh_b, ctiles_i, 0), axis=1, keepdims=True)
    g_nt = jnp.sum(jnp.where(oh_b, ntiles_i, 0), axis=1, keepdims=True)
    g_off = jnp.sum(jnp.where(oh_b, _f32i(offs_row), 0), axis=1, keepdims=True)
    g_cnt = jnp.sum(jnp.where(oh_b, counts_i, 0), axis=1, keepdims=True)
    j = b_col - (g_ct - g_nt)
    be_ref[...] = bexp
    br_ref[...] = jnp.clip(g_off + j * T, 0, NA)
    bn_ref[...] = jnp.clip(g_cnt - j * T, 0, T)


def _moe_kernel(bexp_ref, brow_ref, bn_ref, st_ref,
                h_ref, x1_ref, sw_ref, wg_ref, wu_ref, wd_ref,
                out_ref, hs, acc):
    b = pl.program_id(0)

    @pl.when(b == 0)
    def _init():
        out_ref[...] = x1_ref[...]

    n = bn_ref[b]

    @pl.when(n > 0)
    def _work():
        base = brow_ref[b]

        def gbody(t, _):
            hs[t, :] = h_ref[st_ref[base + t], :]
            return 0
        jax.lax.fori_loop(0, T, gbody, 0, unroll=True)

        hv = hs[...]
        g = _bdot(hv, wg_ref[0])
        u = _bdot(hv, wu_ref[0])
        a = (g * jax.nn.sigmoid(g)) * u
        eo = _bdot(a, wd_ref[0])

        w = sw_ref[pl.ds(base, T), :]
        rows = jax.lax.broadcasted_iota(jnp.int32, (T, 1), 0)
        w = jnp.where(rows < n, w, 0.0)
        acc[...] = eo * w

        def sbody(t, _):
            out_ref[st_ref[base + t], :] += acc[t, :]
            return 0
        jax.lax.fori_loop(0, n, sbody, 0)


def kernel(x, Wq, Wk, Wv, Wo, Wr, Wg, Wu, Wd, w1, w2):
    xf = x.reshape(S, DIM)
    w1r = w1.reshape(1, DIM)
    w2r = w2.reshape(1, DIM)

    q, k, v = pl.pallas_call(
        _qkv_kernel,
        grid=(S // QBLK,),
        in_specs=[
            pl.BlockSpec((QBLK, DIM), lambda i: (i, 0)),
            pl.BlockSpec((1, DIM), lambda i: (0, 0)),
            pl.BlockSpec((DIM, DIM), lambda i: (0, 0)),
            pl.BlockSpec((DIM, DIM), lambda i: (0, 0)),
            pl.BlockSpec((DIM, DIM), lambda i: (0, 0)),
        ],
        out_specs=[pl.BlockSpec((QBLK, DIM), lambda i: (i, 0))] * 3,
        out_shape=[jax.ShapeDtypeStruct((S, DIM), jnp.float32)] * 3,
    )(xf, w1r, Wq, Wk, Wv)

    attn = pl.pallas_call(
        _attn_kernel,
        grid=(HEADS // HG, NAB),
        in_specs=[
            pl.BlockSpec((ABLK, HG * HDIM), lambda h, i: (i, h)),
            pl.BlockSpec((S, HG * HDIM), lambda h, i: (0, h)),
            pl.BlockSpec((S, HG * HDIM), lambda h, i: (0, h)),
        ],
        out_specs=pl.BlockSpec((ABLK, HG * HDIM), lambda h, i: (i, h)),
        out_shape=jax.ShapeDtypeStruct((S, DIM), jnp.float32),
    )(q, k, v)

    x1, h, st, sw, bexp, brow, bn = pl.pallas_call(
        _route_kernel,
        grid=(NSB,),
        in_specs=[
            pl.BlockSpec((SBLK, DIM), lambda i: (i, 0)),
            pl.BlockSpec((SBLK, DIM), lambda i: (i, 0)),
            pl.BlockSpec((DIM, DIM), lambda i: (0, 0)),
            pl.BlockSpec((1, DIM), lambda i: (0, 0)),
            pl.BlockSpec((DIM, E), lambda i: (0, 0)),
        ],
        out_specs=[
            pl.BlockSpec((SBLK, DIM), lambda i: (i, 0)),
            pl.BlockSpec((SBLK, DIM), lambda i: (i, 0)),
            pl.BlockSpec((NA + T, 1), lambda i: (0, 0)),
            pl.BlockSpec((NA + T, 1), lambda i: (0, 0)),
            pl.BlockSpec((NB, 1), lambda i: (0, 0)),
            pl.BlockSpec((NB, 1), lambda i: (0, 0)),
            pl.BlockSpec((NB, 1), lambda i: (0, 0)),
        ],
        out_shape=[
            jax.ShapeDtypeStruct((S, DIM), jnp.float32),
            jax.ShapeDtypeStruct((S, DIM), jnp.float32),
            jax.ShapeDtypeStruct((NA + T, 1), jnp.int32),
            jax.ShapeDtypeStruct((NA + T, 1), jnp.float32),
            jax.ShapeDtypeStruct((NB, 1), jnp.int32),
            jax.ShapeDtypeStruct((NB, 1), jnp.int32),
            jax.ShapeDtypeStruct((NB, 1), jnp.int32),
        ],
        scratch_shapes=[
            pltpu.VMEM((S, 1), jnp.int32),
            pltpu.VMEM((S, 1), jnp.int32),
            pltpu.VMEM((S, 1), jnp.float32),
            pltpu.VMEM((S, 1), jnp.float32),
        ],
    )(xf, attn, Wo, w2r, Wr)

    st = st.reshape(NA + T)
    bexp = bexp.reshape(NB)
    brow = brow.reshape(NB)
    bn = bn.reshape(NB)

    out = pl.pallas_call(
        _moe_kernel,
        grid_spec=pltpu.PrefetchScalarGridSpec(
            num_scalar_prefetch=4,
            grid=(NB,),
            in_specs=[
                pl.BlockSpec((S, DIM), lambda b, *_: (0, 0)),
                pl.BlockSpec((S, DIM), lambda b, *_: (0, 0)),
                pl.BlockSpec((NA + T, 1), lambda b, *_: (0, 0)),
                pl.BlockSpec((1, DIM, INTER), lambda b, be, br, bnn, stt: (be[b], 0, 0)),
                pl.BlockSpec((1, DIM, INTER), lambda b, be, br, bnn, stt: (be[b], 0, 0)),
                pl.BlockSpec((1, INTER, DIM), lambda b, be, br, bnn, stt: (be[b], 0, 0)),
            ],
            out_specs=pl.BlockSpec((S, DIM), lambda b, *_: (0, 0)),
            scratch_shapes=[
                pltpu.VMEM((T, DIM), jnp.float32),
                pltpu.VMEM((T, DIM), jnp.float32),
            ],
        ),
        out_shape=jax.ShapeDtypeStruct((S, DIM), jnp.float32),
        compiler_params=pltpu.CompilerParams(
            dimension_semantics=("arbitrary",),
        ),
    )(bexp, brow, bn, st, h, x1, sw, Wg, Wu, Wd)

    return out.reshape(B, S, DIM)


# speedup vs baseline: 1.0277x; 1.0277x over previous
import jax
import jax.numpy as jnp
from jax.experimental import pallas as pl
from jax.experimental.pallas import tpu as pltpu

B, S, DIM = 1, 2048, 768
HEADS, HDIM = 12, 64
INTER = 1024
E, TOPK = 64, 2
EPS = 1e-6

SBLK = 512
NSB = S // SBLK
QBLK = 512
ABLK = 512
NAB = S // ABLK
T = 128
NA = S * TOPK
NB = NA // T + (E - 1)
CH = 256
NCH = NA // CH


def _rms(x, w):
    return w * (x * jax.lax.rsqrt(jnp.mean(x * x, axis=-1, keepdims=True) + EPS))


def _bdot(a, b):
    return jnp.dot(a.astype(jnp.bfloat16), b.astype(jnp.bfloat16),
                   preferred_element_type=jnp.float32)


def _f32i(x):
    return jnp.rint(x).astype(jnp.int32)


_HI = jax.lax.Precision.HIGHEST


def _qkv_kernel(x_ref, w1_ref, wq_ref, wk_ref, wv_ref, q_ref, k_ref, v_ref):
    xn = _rms(x_ref[...], w1_ref[...])
    q_ref[...] = _bdot(xn, wq_ref[...])
    k_ref[...] = _bdot(xn, wk_ref[...])
    v_ref[...] = _bdot(xn, wv_ref[...])


HG = 6


def _attn_kernel(q_ref, k_ref, v_ref, o_ref):
    for hp in range(HG):
        sl = slice(hp * HDIM, (hp + 1) * HDIM)
        q = (q_ref[:, sl] * (HDIM ** -0.5)).astype(jnp.bfloat16)
        k = k_ref[:, sl].astype(jnp.bfloat16)
        s = jax.lax.dot_general(q, k, (((1,), (1,)), ((), ())),
                                preferred_element_type=jnp.float32)
        m = jnp.max(s, axis=-1, keepdims=True)
        p = jnp.exp(s - m)
        denom = jnp.sum(p, axis=-1, keepdims=True)
        o_ref[:, sl] = _bdot(p, v_ref[:, sl]) / denom


def _route_kernel(x_ref, a_ref, wo_ref, w2_ref, wr_ref,
                  x1_ref, h_ref, st_ref, sw_ref, be_ref, br_ref, bn_ref,
                  i1s, i2s, was, wbs):
    i = pl.program_id(0)
    x1 = x_ref[...] + _bdot(a_ref[...], wo_ref[...])
    x1_ref[...] = x1
    h = _rms(x1, w2_ref[...])
    h_ref[...] = h

    logits = jnp.dot(h, wr_ref[...], preferred_element_type=jnp.float32)
    m = jnp.max(logits, axis=-1, keepdims=True)
    p = jnp.exp(logits - m)
    p = p / jnp.sum(p, axis=-1, keepdims=True)
    v1 = jnp.max(p, axis=-1, keepdims=True)
    i1 = jnp.argmax(p, axis=-1, keepdims=True).astype(jnp.int32)
    cols = jax.lax.broadcasted_iota(jnp.int32, (SBLK, E), 1)
    p2 = jnp.where(cols == i1, -1.0, p)
    v2 = jnp.max(p2, axis=-1, keepdims=True)
    i2 = jnp.argmax(p2, axis=-1, keepdims=True).astype(jnp.int32)
    tot = v1 + v2
    sl = pl.ds(i * SBLK, SBLK)
    i1s[sl, :] = i1
    i2s[sl, :] = i2
    was[sl, :] = v1 / tot
    wbs[sl, :] = v2 / tot

    @pl.when(i == NSB - 1)
    def _dispatch():
        _dispatch_body(i1s, i2s, was, wbs, st_ref, sw_ref,
                       be_ref, br_ref, bn_ref)


def _dispatch_body(i1_ref, i2_ref, wa_ref, wb_ref,
                   st_ref, sw_ref, be_ref, br_ref, bn_ref):
    eid = jnp.concatenate([i1_ref[...], i2_ref[...]], axis=0)
    wt = jnp.concatenate([wa_ref[...], wb_ref[...]], axis=0)
    tokf = jnp.remainder(
        jax.lax.broadcasted_iota(jnp.int32, (NA, 1), 0), S).astype(jnp.float32)

    onehot = (jax.lax.broadcasted_iota(jnp.int32, (NA, E), 1)
              == eid).astype(jnp.float32)
    counts_row = jnp.sum(onehot, axis=0, keepdims=True)

    r_i = jax.lax.broadcasted_iota(jnp.int32, (CH, CH), 0)
    c_i = jax.lax.broadcasted_iota(jnp.int32, (CH, CH), 1)
    stril = (c_i < r_i).astype(jnp.float32)
    carry = jnp.zeros((1, E), jnp.float32)
    rank_chunks = []
    for c in range(NCH):
        oc = onehot[c * CH:(c + 1) * CH, :]
        loc = _bdot(stril, oc)
        rank_chunks.append(jnp.sum((loc + carry) * oc, axis=1, keepdims=True))
        carry = carry + jnp.sum(oc, axis=0, keepdims=True)
    rank = jnp.concatenate(rank_chunks, axis=0)

    ue_r = jax.lax.broadcasted_iota(jnp.int32, (E, E), 0)
    ue_c = jax.lax.broadcasted_iota(jnp.int32, (E, E), 1)
    offs_row = jnp.dot(counts_row, (ue_r < ue_c).astype(jnp.float32),
                       preferred_element_type=jnp.float32, precision=_HI)
    dest = _f32i(jnp.sum(onehot * offs_row, axis=1, keepdims=True) + rank)

    tok_hi = jnp.floor(tokf / 64.0)
    tok_lo = tokf - tok_hi * 64.0
    wt_hi = wt.astype(jnp.bfloat16).astype(jnp.float32)
    wt_lo = wt - wt_hi
    vals = jnp.concatenate([tok_hi, tok_lo, wt_hi, wt_lo], axis=1)
    for c in range(NCH):
        r_row = (jax.lax.broadcasted_iota(jnp.int32, (1, CH), 1) + c * CH)
        mc = (dest == r_row).astype(jnp.float32)
        chunk = jax.lax.dot_general(
            mc.astype(jnp.bfloat16), vals.astype(jnp.bfloat16),
            (((0,), (0,)), ((), ())),
            preferred_element_type=jnp.float32)
        st_ref[c * CH:(c + 1) * CH, :] = _f32i(chunk[:, 0:1] * 64.0
                                               + chunk[:, 1:2])
        sw_ref[c * CH:(c + 1) * CH, :] = chunk[:, 2:3] + chunk[:, 3:4]
    st_ref[NA:, :] = jnp.zeros((T, 1), jnp.int32)
    sw_ref[NA:, :] = jnp.zeros((T, 1), jnp.float32)

    counts_i = _f32i(counts_row)
    ntiles_i = (counts_i + (T - 1)) // T
    ctiles_row = jnp.dot(ntiles_i.astype(jnp.float32),
                         (ue_r <= ue_c).astype(jnp.float32),
                         preferred_element_type=jnp.float32,
                         precision=_HI)
    ctiles_i = _f32i(ctiles_row)
    b_col = jax.lax.broadcasted_iota(jnp.int32, (NB, 1), 0)
    bexp = jnp.minimum(jnp.sum((ctiles_i <= b_col).astype(jnp.int32),
                               axis=1, keepdims=True), E - 1)
    oh_b = (jax.lax.broadcasted_iota(jnp.int32, (NB, E), 1) == bexp)
    g_ct = jnp.sum(jnp.where(oh_b, ctiles_i, 0), axis=1, keepdims=True)
    g_nt = jnp.sum(jnp.where(oh_b, ntiles_i, 0), axis=1, keepdims=True)
    g_off = jnp.sum(jnp.where(oh_b, _f32i(offs_row), 0), axis=1, keepdims=True)
    g_cnt = jnp.sum(jnp.where(oh_b, counts_i, 0), axis=1, keepdims=True)
    j = b_col - (g_ct - g_nt)
    be_ref[...] = bexp
    br_ref[...] = jnp.clip(g_off + j * T, 0, NA)
    bn_ref[...] = jnp.clip(g_cnt - j * T, 0, T)


def _moe_kernel(bexp_ref, brow_ref, bn_ref, st_ref,
                h_ref, x1_ref, sw_ref, wg_ref, wu_ref, wd_ref,
                out_ref, hs, acc):
    b = pl.program_id(0)

    @pl.when(b == 0)
    def _init():
        out_ref[...] = x1_ref[...]

    n = bn_ref[b]

    @pl.when(n > 0)
    def _work():
        base = brow_ref[b]

        def gbody(t, _):
            hs[t, :] = h_ref[st_ref[base + t], :]
            return 0
        jax.lax.fori_loop(0, T, gbody, 0, unroll=True)

        hv = hs[...]
        g = _bdot(hv, wg_ref[0])
        u = _bdot(hv, wu_ref[0])
        a = (g * jax.nn.sigmoid(g)) * u
        eo = _bdot(a, wd_ref[0])

        w = sw_ref[pl.ds(base, T), :]
        rows = jax.lax.broadcasted_iota(jnp.int32, (T, 1), 0)
        w = jnp.where(rows < n, w, 0.0)
        acc[...] = eo * w

        def sbody(t, _):
            out_ref[st_ref[base + t], :] += acc[t, :]
            return 0
        jax.lax.fori_loop(0, n, sbody, 0)


def kernel(x, Wq, Wk, Wv, Wo, Wr, Wg, Wu, Wd, w1, w2):
    xf = x.reshape(S, DIM)
    w1r = w1.reshape(1, DIM)
    w2r = w2.reshape(1, DIM)

    q, k, v = pl.pallas_call(
        _qkv_kernel,
        grid=(S // QBLK,),
        in_specs=[
            pl.BlockSpec((QBLK, DIM), lambda i: (i, 0)),
            pl.BlockSpec((1, DIM), lambda i: (0, 0)),
            pl.BlockSpec((DIM, DIM), lambda i: (0, 0)),
            pl.BlockSpec((DIM, DIM), lambda i: (0, 0)),
            pl.BlockSpec((DIM, DIM), lambda i: (0, 0)),
        ],
        out_specs=[pl.BlockSpec((QBLK, DIM), lambda i: (i, 0))] * 3,
        out_shape=[jax.ShapeDtypeStruct((S, DIM), jnp.float32)] * 3,
    )(xf, w1r, Wq, Wk, Wv)

    attn = pl.pallas_call(
        _attn_kernel,
        grid=(HEADS // HG, NAB),
        in_specs=[
            pl.BlockSpec((ABLK, HG * HDIM), lambda h, i: (i, h)),
            pl.BlockSpec((S, HG * HDIM), lambda h, i: (0, h)),
            pl.BlockSpec((S, HG * HDIM), lambda h, i: (0, h)),
        ],
        out_specs=pl.BlockSpec((ABLK, HG * HDIM), lambda h, i: (i, h)),
        out_shape=jax.ShapeDtypeStruct((S, DIM), jnp.float32),
    )(q, k, v)

    x1, h, st, sw, bexp, brow, bn = pl.pallas_call(
        _route_kernel,
        grid=(NSB,),
        in_specs=[
            pl.BlockSpec((SBLK, DIM), lambda i: (i, 0)),
            pl.BlockSpec((SBLK, DIM), lambda i: (i, 0)),
            pl.BlockSpec((DIM, DIM), lambda i: (0, 0)),
            pl.BlockSpec((1, DIM), lambda i: (0, 0)),
            pl.BlockSpec((DIM, E), lambda i: (0, 0)),
        ],
        out_specs=[
            pl.BlockSpec((SBLK, DIM), lambda i: (i, 0)),
            pl.BlockSpec((SBLK, DIM), lambda i: (i, 0)),
            pl.BlockSpec((NA + T, 1), lambda i: (0, 0)),
            pl.BlockSpec((NA + T, 1), lambda i: (0, 0)),
            pl.BlockSpec((NB, 1), lambda i: (0, 0)),
            pl.BlockSpec((NB, 1), lambda i: (0, 0)),
            pl.BlockSpec((NB, 1), lambda i: (0, 0)),
        ],
        out_shape=[
            jax.ShapeDtypeStruct((S, DIM), jnp.float32),
            jax.ShapeDtypeStruct((S, DIM), jnp.float32),
            jax.ShapeDtypeStruct((NA + T, 1), jnp.int32),
            jax.ShapeDtypeStruct((NA + T, 1), jnp.float32),
            jax.ShapeDtypeStruct((NB, 1), jnp.int32),
            jax.ShapeDtypeStruct((NB, 1), jnp.int32),
            jax.ShapeDtypeStruct((NB, 1), jnp.int32),
        ],
        scratch_shapes=[
            pltpu.VMEM((S, 1), jnp.int32),
            pltpu.VMEM((S, 1), jnp.int32),
            pltpu.VMEM((S, 1), jnp.float32),
            pltpu.VMEM((S, 1), jnp.float32),
        ],
    )(xf, attn, Wo, w2r, Wr)

    st = st.reshape(NA + T)
    bexp = bexp.reshape(NB)
    brow = brow.reshape(NB)
    bn = bn.reshape(NB)

    out = pl.pallas_call(
        _moe_kernel,
        grid_spec=pltpu.PrefetchScalarGridSpec(
            num_scalar_prefetch=4,
            grid=(NB,),
            in_specs=[
                pl.BlockSpec((S, DIM), lambda b, *_: (0, 0)),
                pl.BlockSpec((S, DIM), lambda b, *_: (0, 0)),
                pl.BlockSpec((NA + T, 1), lambda b, *_: (0, 0)),
                pl.BlockSpec((1, DIM, INTER), lambda b, be, br, bnn, stt: (be[b], 0, 0)),
                pl.BlockSpec((1, DIM, INTER), lambda b, be, br, bnn, stt: (be[b], 0, 0)),
                pl.BlockSpec((1, INTER, DIM), lambda b, be, br, bnn, stt: (be[b], 0, 0)),
            ],
            out_specs=pl.BlockSpec((S, DIM), lambda b, *_: (0, 0)),
            scratch_shapes=[
                pltpu.VMEM((T, DIM), jnp.float32),
                pltpu.VMEM((T, DIM), jnp.float32),
            ],
        ),
        out_shape=jax.ShapeDtypeStruct((S, DIM), jnp.float32),
        compiler_params=pltpu.CompilerParams(
            dimension_semantics=("arbitrary",),
        ),
    )(bexp, brow, bn, st, h, x1, sw, Wg, Wu, Wd)

    return out.reshape(B, S, DIM)
